# baseline (speedup 1.0000x reference)
import functools

import jax
import jax.numpy as jnp
from jax import lax
from jax.experimental import pallas as pl
from jax.experimental.pallas import tpu as pltpu
from jax.experimental.pallas import tpu_sc as plsc

N_ATOMS_C = 320000
D = 128
NL = 16
NVPR = D // NL
NSEG = 1024
NC = 2
NS = 16
NW = NC * NS
PER_TILE = N_ATOMS_C // NW
CH = 80
NCHUNK = PER_TILE // CH
NBUF = 5
STG = 192
FLUSH_AT = 96
ACC_ROWS = 1216
ZROWS = ACC_ROWS // NS
ROWS_PER_TILE = NSEG // NS

_mesh = plsc.VectorSubcoreMesh(core_axis_name="c", subcore_axis_name="s")


@functools.partial(
    pl.kernel,
    mesh=_mesh,
    out_type=jax.ShapeDtypeStruct((NC * NSEG, D), jnp.float32),
    scratch_types=(
        [pltpu.VMEM((PER_TILE + NL,), jnp.int32)]
        + [pltpu.VMEM((CH, D), jnp.float32) for _ in range(NBUF)]
        + [pltpu.VMEM((STG, D), jnp.float32)]
        + [pltpu.VMEM((STG,), jnp.int32)]
        + [pltpu.VMEM_SHARED((ACC_ROWS, D), jnp.float32)]
        + [pltpu.SMEM((1,), jnp.int32)]
        + [pltpu.SemaphoreType.DMA for _ in range(NBUF)]
    ),
)
def _segment_sum_sc(feat_hbm, ids_hbm, out_hbm, *refs):
    ids_v = refs[0]
    rows_bufs = refs[1:NBUF + 1]
    stage_rows = refs[NBUF + 1]
    stage_ids = refs[NBUF + 2]
    acc_sh = refs[NBUF + 3]
    k_ref = refs[NBUF + 4]
    sem_r = refs[NBUF + 5:2 * NBUF + 5]
    cid = lax.axis_index("c")
    sid = lax.axis_index("s")
    wid = cid * NS + sid
    base_row = wid * PER_TILE

    pltpu.sync_copy(ids_hbm.at[pl.ds(base_row, PER_TILE)],
                    ids_v.at[pl.ds(0, PER_TILE)])

    def start_load(c, b):
        pltpu.make_async_copy(
            feat_hbm.at[pl.ds(base_row + c * CH, CH)],
            rows_bufs[b], sem_r[b]).start()

    def wait_load(b):
        pltpu.make_async_copy(
            feat_hbm.at[pl.ds(0, CH)], rows_bufs[b], sem_r[b]).wait()

    trash16 = jnp.full((NL,), NSEG, jnp.int32)
    zero16 = jnp.zeros((NL,), jnp.float32)

    def zero_stage():
        def zb(i, carry):
            stage_rows[i // NVPR, pl.ds((i % NVPR) * NL, NL)] = zero16
            return carry
        lax.fori_loop(0, STG * NVPR, zb, 0)
        for t in range(STG // NL):
            stage_ids[pl.ds(t * NL, NL)] = trash16

    zero_stage()
    k_ref[0] = 0

    def zero_body(i, carry):
        rows_bufs[0][i // NVPR, pl.ds((i % NVPR) * NL, NL)] = zero16
        return carry

    lax.fori_loop(0, ZROWS * NVPR, zero_body, 0)
    pltpu.sync_copy(rows_bufs[0].at[pl.ds(0, ZROWS)],
                    acc_sh.at[pl.ds(sid * ZROWS, ZROWS)])
    plsc.subcore_barrier()

    for b in range(NBUF):
        start_load(b, b)

    def drain():
        stage_ids[pl.ds(k_ref[0] + 1, NL)] = trash16
        pltpu.sync_copy(stage_rows, acc_sh.at[stage_ids], add=True)
        zero_stage()
        k_ref[0] = 0

    def group_body(i, carry):
        g = i * NBUF
        for b in range(NBUF):
            c = g + b
            wait_load(b)

            def row_body(r, prev_vec):
                id_vec = ids_v[pl.ds(c * CH + r, NL)]

                @pl.when(id_vec[0] != prev_vec[0])
                def _():
                    k_ref[0] = k_ref[0] + 1

                kk = k_ref[0]
                for j in range(NVPR):
                    plsc.addupdate(stage_rows.at[kk, pl.ds(j * NL, NL)],
                                   rows_bufs[b][r, pl.ds(j * NL, NL)])
                stage_ids[pl.ds(kk, NL)] = id_vec
                return id_vec

            carry = lax.fori_loop(0, CH, row_body, carry)

            @pl.when(k_ref[0] >= FLUSH_AT)
            def _():
                drain()

            start_load(jnp.minimum(c + NBUF, NCHUNK - 1), b)
        return carry

    lax.fori_loop(0, NCHUNK // NBUF, group_body, ids_v[pl.ds(0, NL)])
    drain()

    for b in range(NBUF):
        wait_load(b)
    plsc.subcore_barrier()

    pltpu.sync_copy(
        acc_sh.at[pl.ds(sid * ROWS_PER_TILE, ROWS_PER_TILE)],
        out_hbm.at[pl.ds(cid * NSEG + sid * ROWS_PER_TILE, ROWS_PER_TILE)])


def _add2_body(a_ref, b_ref, o_ref):
    o_ref[...] = a_ref[...] + b_ref[...]


def kernel(atom_features, atom_split):
    ids = atom_split.astype(jnp.int32)
    partial = _segment_sum_sc(atom_features, ids)
    return pl.pallas_call(
        _add2_body,
        out_shape=jax.ShapeDtypeStruct((NSEG, D), jnp.float32),
    )(partial[:NSEG], partial[NSEG:])


# speedup vs baseline: 2.2613x; 2.2613x over previous
import functools

import jax
import jax.numpy as jnp
from jax import lax
from jax.experimental import pallas as pl
from jax.experimental.pallas import tpu as pltpu
from jax.experimental.pallas import tpu_sc as plsc

N_ATOMS_C = 320000
D = 128
NL = 16
NVPR = D // NL
NSEG = 1024
NC = 2
NS = 16
NW = NC * NS
PER_TILE = N_ATOMS_C // NW
CH = 80
NCHUNK = PER_TILE // CH
NBUF = 5
STG = 192
FLUSH_AT = 96
ACC_ROWS = 1216
ZROWS = ACC_ROWS // NS
ROWS_PER_TILE = NSEG // NS

_mesh = plsc.VectorSubcoreMesh(core_axis_name="c", subcore_axis_name="s")


@functools.partial(
    pl.kernel,
    mesh=_mesh,
    out_type=jax.ShapeDtypeStruct((NC * NSEG, D), jnp.float32),
    scratch_types=(
        [pltpu.VMEM((PER_TILE + NL,), jnp.int32)]
        + [pltpu.VMEM((CH, D), jnp.float32) for _ in range(NBUF)]
        + [pltpu.VMEM((STG, D), jnp.float32)]
        + [pltpu.VMEM((STG,), jnp.int32)]
        + [pltpu.VMEM_SHARED((ACC_ROWS, D), jnp.float32)]
        + [pltpu.SMEM((1,), jnp.int32)]
        + [pltpu.SemaphoreType.DMA for _ in range(NBUF)]
    ),
)
def _segment_sum_sc(feat_hbm, ids_hbm, out_hbm, *refs):
    ids_v = refs[0]
    rows_bufs = refs[1:NBUF + 1]
    stage_rows = refs[NBUF + 1]
    stage_ids = refs[NBUF + 2]
    acc_sh = refs[NBUF + 3]
    k_ref = refs[NBUF + 4]
    sem_r = refs[NBUF + 5:2 * NBUF + 5]
    cid = lax.axis_index("c")
    sid = lax.axis_index("s")
    wid = cid * NS + sid
    base_row = wid * PER_TILE

    pltpu.sync_copy(ids_hbm.at[pl.ds(base_row, PER_TILE)],
                    ids_v.at[pl.ds(0, PER_TILE)])

    def start_load(c, b):
        pltpu.make_async_copy(
            feat_hbm.at[pl.ds(base_row + c * CH, CH)],
            rows_bufs[b], sem_r[b]).start()

    def wait_load(b):
        pltpu.make_async_copy(
            feat_hbm.at[pl.ds(0, CH)], rows_bufs[b], sem_r[b]).wait()

    trash16 = jnp.full((NL,), NSEG, jnp.int32)
    zero16 = jnp.zeros((NL,), jnp.float32)

    def zero_stage():
        def zb(i, carry):
            stage_rows[i // NVPR, pl.ds((i % NVPR) * NL, NL)] = zero16
            return carry
        lax.fori_loop(0, STG * NVPR, zb, 0)
        for t in range(STG // NL):
            stage_ids[pl.ds(t * NL, NL)] = trash16

    zero_stage()
    k_ref[0] = 0

    def zero_body(i, carry):
        rows_bufs[0][i // NVPR, pl.ds((i % NVPR) * NL, NL)] = zero16
        return carry

    lax.fori_loop(0, ZROWS * NVPR, zero_body, 0)
    pltpu.sync_copy(rows_bufs[0].at[pl.ds(0, ZROWS)],
                    acc_sh.at[pl.ds(sid * ZROWS, ZROWS)])
    plsc.subcore_barrier()

    for b in range(NBUF):
        start_load(b, b)

    def drain():
        stage_ids[pl.ds(k_ref[0] + 1, NL)] = trash16
        pltpu.sync_copy(stage_rows, acc_sh.at[stage_ids], add=True)
        zero_stage()
        k_ref[0] = 0

    def group_body(i, carry):
        g = i * NBUF
        for b in range(NBUF):
            c = g + b
            wait_load(b)

            def win_body(w, prev_vec):
                r0 = w * NL
                id_vec = ids_v[pl.ds(c * CH + r0, NL)]
                uniform = id_vec[0] == id_vec[NL - 1]
                starts_new = id_vec[0] != prev_vec[0]

                @pl.when(uniform)
                def _():
                    @pl.when(starts_new)
                    def _():
                        k_ref[0] = k_ref[0] + 1

                    kk = k_ref[0]
                    for j in range(NVPR):
                        acc = rows_bufs[b][r0, pl.ds(j * NL, NL)]
                        for rr in range(1, NL):
                            acc = acc + rows_bufs[b][r0 + rr,
                                                     pl.ds(j * NL, NL)]
                        plsc.addupdate(stage_rows.at[kk, pl.ds(j * NL, NL)],
                                       acc)
                    stage_ids[pl.ds(kk, NL)] = id_vec

                @pl.when(jnp.logical_not(uniform))
                def _():
                    def row_body(r, pv):
                        rid = ids_v[pl.ds(c * CH + r, NL)]

                        @pl.when(rid[0] != pv[0])
                        def _():
                            k_ref[0] = k_ref[0] + 1

                        kk = k_ref[0]
                        for j in range(NVPR):
                            plsc.addupdate(
                                stage_rows.at[kk, pl.ds(j * NL, NL)],
                                rows_bufs[b][r, pl.ds(j * NL, NL)])
                        stage_ids[pl.ds(kk, NL)] = rid
                        return rid

                    lax.fori_loop(r0, r0 + NL, row_body, prev_vec)

                return ids_v[pl.ds(c * CH + r0 + NL - 1, NL)]

            carry = lax.fori_loop(0, CH // NL, win_body, carry)

            @pl.when(k_ref[0] >= FLUSH_AT)
            def _():
                drain()

            start_load(jnp.minimum(c + NBUF, NCHUNK - 1), b)
        return carry

    lax.fori_loop(0, NCHUNK // NBUF, group_body, ids_v[pl.ds(0, NL)])
    drain()

    for b in range(NBUF):
        wait_load(b)
    plsc.subcore_barrier()

    pltpu.sync_copy(
        acc_sh.at[pl.ds(sid * ROWS_PER_TILE, ROWS_PER_TILE)],
        out_hbm.at[pl.ds(cid * NSEG + sid * ROWS_PER_TILE, ROWS_PER_TILE)])


def _add2_body(a_ref, b_ref, o_ref):
    o_ref[...] = a_ref[...] + b_ref[...]


def kernel(atom_features, atom_split):
    ids = atom_split.astype(jnp.int32)
    partial = _segment_sum_sc(atom_features, ids)
    return pl.pallas_call(
        _add2_body,
        out_shape=jax.ShapeDtypeStruct((NSEG, D), jnp.float32),
    )(partial[:NSEG], partial[NSEG:])


# speedup vs baseline: 3.2501x; 1.4372x over previous
import functools

import jax
import jax.numpy as jnp
from jax import lax
from jax.experimental import pallas as pl
from jax.experimental.pallas import tpu as pltpu
from jax.experimental.pallas import tpu_sc as plsc

N_ATOMS_C = 320000
D = 128
NSEG = 1024
NC = 2
NS = 16
NW = NC * NS
PER_TILE = N_ATOMS_C // NW
CH = 80
NCHUNK = PER_TILE // CH
NBUF = 5
ROWS_PER_TILE = NSEG // NS

_mesh = plsc.VectorSubcoreMesh(core_axis_name="c", subcore_axis_name="s")


@functools.partial(
    pl.kernel,
    mesh=_mesh,
    out_type=jax.ShapeDtypeStruct((NC * NSEG, D), jnp.float32),
    scratch_types=(
        [pltpu.VMEM((CH,), jnp.int32) for _ in range(NBUF)]
        + [pltpu.VMEM((CH, D), jnp.float32) for _ in range(NBUF)]
        + [pltpu.VMEM_SHARED((NSEG, D), jnp.float32)]
        + [pltpu.SemaphoreType.DMA for _ in range(2 * NBUF)]
    ),
)
def _segment_sum_sc(feat_hbm, ids_hbm, out_hbm, *refs):
    ids_bufs = refs[0:NBUF]
    rows_bufs = refs[NBUF:2 * NBUF]
    acc_sh = refs[2 * NBUF]
    sem_i = refs[2 * NBUF + 1:3 * NBUF + 1]
    sem_r = refs[3 * NBUF + 1:4 * NBUF + 1]
    cid = lax.axis_index("c")
    sid = lax.axis_index("s")
    wid = cid * NS + sid
    base_row = wid * PER_TILE

    def start_load(c, b):
        off = base_row + c * CH
        pltpu.make_async_copy(
            ids_hbm.at[pl.ds(off, CH)], ids_bufs[b], sem_i[b]).start()
        pltpu.make_async_copy(
            feat_hbm.at[pl.ds(off, CH)], rows_bufs[b], sem_r[b]).start()

    def wait_load(b):
        pltpu.make_async_copy(
            ids_hbm.at[pl.ds(0, CH)], ids_bufs[b], sem_i[b]).wait()
        pltpu.make_async_copy(
            feat_hbm.at[pl.ds(0, CH)], rows_bufs[b], sem_r[b]).wait()

    zero16 = jnp.zeros((16,), jnp.float32)

    def zero_body(i, carry):
        r = i // (D // 16)
        j = i % (D // 16)
        rows_bufs[0][r, pl.ds(j * 16, 16)] = zero16
        return carry

    lax.fori_loop(0, ROWS_PER_TILE * (D // 16), zero_body, 0)
    pltpu.sync_copy(rows_bufs[0].at[pl.ds(0, ROWS_PER_TILE)],
                    acc_sh.at[pl.ds(sid * ROWS_PER_TILE, ROWS_PER_TILE)])
    plsc.subcore_barrier()

    for b in range(NBUF):
        start_load(b, b)

    def group_body(i, carry):
        g = i * NBUF
        for b in range(NBUF):
            c = g + b
            wait_load(b)
            pltpu.sync_copy(rows_bufs[b], acc_sh.at[ids_bufs[b]], add=True)
            start_load(jnp.minimum(c + NBUF, NCHUNK - 1), b)
        return carry

    lax.fori_loop(0, NCHUNK // NBUF, group_body, 0)
    for b in range(NBUF):
        wait_load(b)
    plsc.subcore_barrier()

    pltpu.sync_copy(
        acc_sh.at[pl.ds(sid * ROWS_PER_TILE, ROWS_PER_TILE)],
        out_hbm.at[pl.ds(cid * NSEG + sid * ROWS_PER_TILE, ROWS_PER_TILE)])


def _add2_body(a_ref, b_ref, o_ref):
    o_ref[...] = a_ref[...] + b_ref[...]


def kernel(atom_features, atom_split):
    ids = atom_split.astype(jnp.int32)
    partial = _segment_sum_sc(atom_features, ids)
    return pl.pallas_call(
        _add2_body,
        out_shape=jax.ShapeDtypeStruct((NSEG, D), jnp.float32),
    )(partial[:NSEG], partial[NSEG:])
